# baseline (speedup 1.0000x reference)
import functools

import jax
import jax.numpy as jnp
from jax import lax
from jax.experimental import pallas as pl
from jax.experimental.pallas import tpu as pltpu
from jax.experimental.pallas import tpu_sc as plsc

_D = 64
_IDX_ROW = 128


@functools.cache
def _make_sc_gather(n_total, nc, ns, rows_per_chunk, n_chunks):
    nw = nc * ns
    chunk_elems = rows_per_chunk * _IDX_ROW
    mesh = plsc.VectorSubcoreMesh(core_axis_name="c", subcore_axis_name="s")

    @functools.partial(
        pl.kernel,
        mesh=mesh,
        out_type=jax.ShapeDtypeStruct((n_total, _D), jnp.float32),
        scratch_types=[
            pltpu.VMEM((rows_per_chunk, _IDX_ROW), jnp.int32),
            pltpu.VMEM((chunk_elems, _D), jnp.float32),
            pltpu.SemaphoreType.DMA,
        ],
        compiler_params=pltpu.CompilerParams(use_tc_tiling_on_sc=False),
    )
    def gather_kernel(w_hbm, idx_hbm, out_hbm, idx_v, rows_v, sem):
        wid = lax.axis_index("s") * nc + lax.axis_index("c")
        row0 = wid * (n_chunks * rows_per_chunk)

        def chunk_body(i, carry):
            r0 = row0 + i * rows_per_chunk
            pltpu.sync_copy(idx_hbm.at[pl.ds(r0, rows_per_chunk)], idx_v)
            copies = [
                pltpu.async_copy(
                    w_hbm.at[idx_v.at[j]],
                    rows_v.at[pl.ds(j * _IDX_ROW, _IDX_ROW)],
                    sem,
                )
                for j in range(rows_per_chunk)
            ]
            for c in copies:
                c.wait()
            pltpu.sync_copy(
                rows_v, out_hbm.at[pl.ds(r0 * _IDX_ROW, chunk_elems)]
            )
            return carry

        lax.fori_loop(0, n_chunks, chunk_body, 0)

    return gather_kernel


def kernel(x, W):
    n_total = x.size
    info = plsc.get_sparse_core_info()
    nc, ns = info.num_cores, info.num_subcores
    nw = nc * ns
    n_idx_rows = n_total // _IDX_ROW
    rows_per_worker = n_idx_rows // nw
    rows_per_chunk = 8
    n_chunks = rows_per_worker // rows_per_chunk
    idx2d = x.reshape(n_idx_rows, _IDX_ROW)
    fn = _make_sc_gather(n_total, nc, ns, rows_per_chunk, n_chunks)
    out = fn(W, idx2d)
    return out.reshape(*x.shape, _D)


# speedup vs baseline: 1.0182x; 1.0182x over previous
import functools

import jax
import jax.numpy as jnp
from jax import lax
from jax.experimental import pallas as pl
from jax.experimental.pallas import tpu as pltpu
from jax.experimental.pallas import tpu_sc as plsc

_D = 64
_IDX_ROW = 128


@functools.cache
def _make_sc_gather(n_total, nc, ns, rows_per_chunk, n_chunks):
    chunk_elems = rows_per_chunk * _IDX_ROW
    n_outer = n_chunks // 2
    mesh = plsc.VectorSubcoreMesh(core_axis_name="c", subcore_axis_name="s")

    @functools.partial(
        pl.kernel,
        mesh=mesh,
        out_type=jax.ShapeDtypeStruct((n_total, _D), jnp.float32),
        scratch_types=[
            pltpu.VMEM((2, rows_per_chunk, _IDX_ROW), jnp.int32),
            pltpu.VMEM((2, chunk_elems, _D), jnp.float32),
            pltpu.SemaphoreType.DMA,
            pltpu.SemaphoreType.DMA,
            pltpu.SemaphoreType.DMA,
            pltpu.SemaphoreType.DMA,
            pltpu.SemaphoreType.DMA,
            pltpu.SemaphoreType.DMA,
        ],
        compiler_params=pltpu.CompilerParams(use_tc_tiling_on_sc=False),
    )
    def gather_kernel(w_hbm, idx_hbm, out_hbm, idx_v, rows_v,
                      is0, is1, gs0, gs1, os0, os1):
        wid = lax.axis_index("s") * nc + lax.axis_index("c")
        row0 = wid * (n_chunks * rows_per_chunk)
        isems = (is0, is1)
        gsems = (gs0, gs1)
        osems = (os0, os1)

        def idx_copy(g, b):
            return pltpu.make_async_copy(
                idx_hbm.at[pl.ds(row0 + g * rows_per_chunk, rows_per_chunk)],
                idx_v.at[b],
                isems[b],
            )

        def out_copy(g, b):
            return pltpu.make_async_copy(
                rows_v.at[b],
                out_hbm.at[pl.ds((row0 + g * rows_per_chunk) * _IDX_ROW,
                                 chunk_elems)],
                osems[b],
            )

        idx_copy(0, 0).start()
        idx_copy(1, 1).start()

        def outer_body(o, carry):
            for b in range(2):
                g = 2 * o + b

                @pl.when(o > 0)
                def _():
                    out_copy(g - 2, b).wait()

                idx_copy(g, b).wait()
                gathers = [
                    pltpu.async_copy(
                        w_hbm.at[idx_v.at[b].at[j]],
                        rows_v.at[b].at[pl.ds(j * _IDX_ROW, _IDX_ROW)],
                        gsems[b],
                    )
                    for j in range(rows_per_chunk)
                ]
                for c in gathers:
                    c.wait()

                @pl.when(o < n_outer - 1)
                def _():
                    idx_copy(g + 2, b).start()

                out_copy(g, b).start()
            return carry

        lax.fori_loop(0, n_outer, outer_body, 0)
        out_copy(n_chunks - 2, 0).wait()
        out_copy(n_chunks - 1, 1).wait()

    return gather_kernel


def kernel(x, W):
    n_total = x.size
    info = plsc.get_sparse_core_info()
    nc, ns = info.num_cores, info.num_subcores
    nw = nc * ns
    n_idx_rows = n_total // _IDX_ROW
    rows_per_worker = n_idx_rows // nw
    rows_per_chunk = 5
    n_chunks = rows_per_worker // rows_per_chunk
    idx2d = x.reshape(n_idx_rows, _IDX_ROW)
    fn = _make_sc_gather(n_total, nc, ns, rows_per_chunk, n_chunks)
    out = fn(W, idx2d)
    return out.reshape(*x.shape, _D)
